# baseline (speedup 1.0000x reference)
import functools

import jax
import jax.numpy as jnp
from jax import lax
from jax.experimental import pallas as pl
from jax.experimental.pallas import tpu as pltpu
from jax.experimental.pallas import tpu_sc as plsc

_N = 10000
_H = 64
_G = 16
_SUB = 128
_NC = 2
_NS = 16
_NW = _NC * _NS
_NPAD = 10240
_NSH = _NPAD // _NS


def _seg_inner(rpw0, rpw1, p_hbm, src_hbm, dst_hbm,
               seg_out, src_idx, dst_idx, gbuf0, gbuf1, gbuf2, gbuf3, acc,
               semg0, semg1, semg2, semg3, sems0, sems1, sems2, sems3):
    c = lax.axis_index("c")
    s = lax.axis_index("s")
    rpw = jnp.where(c == 0, rpw0, rpw1)
    base = pl.multiple_of(
        jnp.where(c == 0, s * rpw0, _NS * rpw0 + s * rpw1), 8)
    rmax = max(rpw0, rpw1)
    pltpu.sync_copy(src_hbm.at[pl.ds(base, rmax)], src_idx)
    pltpu.sync_copy(dst_hbm.at[pl.ds(base, rmax)], dst_idx)
    zv = jnp.zeros((16,), jnp.float32)

    def zrow(i, carry):
        for j in range(_H // 16):
            gbuf0[i, pl.ds(j * 16, 16)] = zv
        return carry

    lax.fori_loop(0, _SUB, zrow, 0)
    for i in range(_NSH // _SUB):
        pltpu.sync_copy(gbuf0, acc.at[pl.ds(s * _NSH + i * _SUB, _SUB)])
    plsc.subcore_barrier()

    gbufs = (gbuf0, gbuf1, gbuf2, gbuf3)
    semg = (semg0, semg1, semg2, semg3)
    sems = (sems0, sems1, sems2, sems3)
    dummy = p_hbm.at[pl.ds(0, _SUB)]

    def g_start(k, b):
        pltpu.async_copy(p_hbm.at[src_idx.at[k]], gbufs[b], semg[b])

    def g_wait(b):
        pltpu.make_async_copy(dummy, gbufs[b], semg[b]).wait()

    def s_start(k, b):
        pltpu.async_copy(gbufs[b], acc.at[dst_idx.at[k]], sems[b], add=True)

    def s_wait(b):
        pltpu.make_async_copy(dummy, gbufs[b], sems[b]).wait()

    def ops(k, b, do_swait, do_gstart):
        b2 = (b + 2) % 4
        g_wait(b)
        s_start(k, b)
        if do_swait:
            s_wait(b2)
        if do_gstart:
            g_start(k + 2, b2)

    g_start(0, 0)
    g_start(1, 1)
    ops(0, 0, False, False)
    g_start(2, 2)
    ops(1, 1, False, False)
    g_start(3, 3)
    ops(2, 2, True, True)
    ops(3, 3, True, True)

    def round_(j, carry):
        k = 4 + 4 * j
        ops(k, 0, True, True)
        ops(k + 1, 1, True, True)
        ops(k + 2, 2, True, True)
        ops(k + 3, 3, True, True)
        return carry

    lax.fori_loop(0, (rpw - 8) // 4, round_, 0)

    ops(rpw - 4, 0, True, True)
    ops(rpw - 3, 1, True, True)
    ops(rpw - 2, 2, True, False)
    ops(rpw - 1, 3, True, False)
    s_wait(2)
    s_wait(3)

    plsc.subcore_barrier()
    sl = pl.ds(s * _NSH, _NSH)
    pltpu.sync_copy(acc.at[sl], seg_out.at[c, sl])


@functools.lru_cache(maxsize=None)
def _make_seg(rpw0, rpw1):
    mesh = plsc.VectorSubcoreMesh(core_axis_name="c", subcore_axis_name="s")
    out_type = [jax.ShapeDtypeStruct((_NC, _NPAD, _H), jnp.float32)]
    rmax = max(rpw0, rpw1)
    scratch = (
        [pltpu.VMEM((rmax, _SUB), jnp.int32)] * 2
        + [pltpu.VMEM((_SUB, _H), jnp.float32)] * 4
        + [pltpu.VMEM_SHARED((_NPAD, _H), jnp.float32)]
        + [pltpu.SemaphoreType.DMA] * 8
    )

    @functools.partial(pl.kernel, out_type=out_type, mesh=mesh,
                       compiler_params=pltpu.CompilerParams(
                           use_tc_tiling_on_sc=False),
                       scratch_types=scratch)
    def seg_k(p_hbm, src_hbm, dst_hbm, seg_out,
              src_idx, dst_idx, gbuf0, gbuf1, gbuf2, gbuf3, acc,
              semg0, semg1, semg2, semg3, sems0, sems1, sems2, sems3):
        _seg_inner(rpw0, rpw1, p_hbm, src_hbm, dst_hbm,
                   seg_out, src_idx, dst_idx,
                   gbuf0, gbuf1, gbuf2, gbuf3, acc,
                   semg0, semg1, semg2, semg3,
                   sems0, sems1, sems2, sems3)

    return seg_k


@functools.lru_cache(maxsize=None)
def _make_cnt(rpw):
    mesh = plsc.VectorSubcoreMesh(core_axis_name="c", subcore_axis_name="s")

    @functools.partial(
        pl.kernel,
        out_type=[jax.ShapeDtypeStruct((_NC, _NPAD, 8), jnp.float32)],
        mesh=mesh,
        compiler_params=pltpu.CompilerParams(use_tc_tiling_on_sc=False),
        scratch_types=[
            pltpu.VMEM((rpw, _SUB), jnp.int32),
            pltpu.VMEM((_SUB, 8), jnp.float32),
            pltpu.VMEM_SHARED((_NPAD, 8), jnp.float32),
            pltpu.SemaphoreType.DMA,
        ])
    def cnt_k(dst_hbm, zc_hbm, ones_hbm, cnt_out,
              dst_idx, ones_v, cnt_acc, semc):
        c = lax.axis_index("c")
        s = lax.axis_index("s")
        base = (c * _NS + s) * rpw
        pltpu.sync_copy(dst_hbm.at[pl.ds(base, rpw)], dst_idx)
        psl = pl.ds(s * _NSH, _NSH)
        pltpu.sync_copy(zc_hbm, cnt_acc.at[psl])
        pltpu.sync_copy(ones_hbm, ones_v)
        plsc.subcore_barrier()

        def step(k, carry):
            pltpu.async_copy(ones_v, cnt_acc.at[dst_idx.at[k]], semc,
                             add=True)
            return carry

        lax.fori_loop(0, rpw, step, 0)

        def drain(i, carry):
            pltpu.make_async_copy(ones_hbm, ones_v, semc).wait()
            return carry

        lax.fori_loop(0, rpw, drain, 0)
        plsc.subcore_barrier()
        pltpu.sync_copy(cnt_acc.at[psl], cnt_out.at[c, psl])

    return cnt_k


_DNUM = (((1,), (1,)), ((), ()))


def _tc_a_body(x_ref, wl_ref, wr_ref, p_out, r_out):
    xv = x_ref[...]
    p_out[...] = lax.dot_general(xv, wl_ref[...], _DNUM,
                                 preferred_element_type=jnp.float32)
    r_out[...] = lax.dot_general(xv, wr_ref[...], _DNUM,
                                 preferred_element_type=jnp.float32)


def _tc_mid_body(segp_ref, cntp_ref, r_ref, blc_ref, wl_ref, wr_ref,
                 bln_ref, flag_ref, p_out, r_out, h_out):
    seg = segp_ref[0] + segp_ref[1]
    cnt = cntp_ref[0, :, 0:1] + cntp_ref[1, :, 0:1]
    out = seg / jnp.maximum(cnt, 1.0) + blc_ref[...] + r_ref[...]
    f = flag_ref[...]
    h = out + f * (jnp.maximum(out, 0.0) - out)
    h_out[...] = h
    p_out[...] = lax.dot_general(h, wl_ref[...], _DNUM,
                                 preferred_element_type=jnp.float32)
    r_out[...] = lax.dot_general(h, wr_ref[...], _DNUM,
                                 preferred_element_type=jnp.float32) + bln_ref[...]


def _tc_c_body(h_ref, wm1_ref, bm1_ref, wm2_ref, bm2_ref,
               batch_ref, bt_ref, out_ref):
    h = h_ref[...]
    m = jnp.maximum(lax.dot_general(h, wm1_ref[...], _DNUM,
                                    preferred_element_type=jnp.float32)
                    + bm1_ref[...], 0.0)
    z = jnp.sum(m * wm2_ref[...], axis=1, keepdims=True) + bm2_ref[...]
    pi = jax.nn.sigmoid(z)
    b = batch_ref[...]
    gid = lax.broadcasted_iota(jnp.int32, (1, _G), 1)
    onehot = (b == gid).astype(jnp.float32)
    total = jnp.sum(onehot * pi, axis=0, keepdims=True)
    ratio = jnp.minimum(bt_ref[...] / (total + 1e-12), 1.0)
    rnode = jnp.sum(onehot * ratio, axis=1, keepdims=True)
    out_ref[...] = pi * rnode


def _sds(*shape):
    return jax.ShapeDtypeStruct(shape, jnp.float32)


@jax.jit
def _impl(x, edge_index, batch, B_total,
          Wl1, bl1, Wr1, Wl2, bl2, Wr2, Wm1, bm1, Wm2, bm2):
    n, f_in = x.shape
    e = edge_index.shape[1]
    rpw = -(-e // (_NW * _SUB))
    rpw = -(-rpw // 8) * 8
    rt = rpw * _NW
    epad = rt * _SUB
    src_p = jnp.concatenate(
        [edge_index[0], jnp.zeros((epad - e,), jnp.int32)]).reshape(rt, _SUB)
    dst_p = jnp.concatenate(
        [edge_index[1], jnp.full((epad - e,), _N, jnp.int32)]).reshape(rt, _SUB)
    zc = jnp.zeros((_NSH, 8), jnp.float32)
    ones8 = jnp.ones((_SUB, 8), jnp.float32)
    x_p = jnp.pad(x, ((0, _NPAD - n), (0, 0)))
    batch_p = jnp.concatenate(
        [batch, jnp.full((_NPAD - n,), _G, jnp.int32)])

    p1, r1 = pl.pallas_call(
        _tc_a_body,
        out_shape=[_sds(_NPAD, _H), _sds(_NPAD, _H)],
    )(x_p, Wl1, Wr1)

    (cntp,) = _make_cnt(rpw)(dst_p, zc, ones8)
    rpw0 = max(8, (2 * rpw * 3 // 10) // 8 * 8)
    rpw1 = 2 * rpw - rpw0
    seg_fn = _make_seg(rpw0, rpw1)

    stack_wl = jnp.stack([Wl2, jnp.zeros_like(Wl2)])
    stack_wr = jnp.stack([Wr2, jnp.zeros_like(Wr2)])
    stack_blc = jnp.stack([bl1, bl2]).reshape(2, 1, _H)
    stack_bln = jnp.stack([bl2, jnp.zeros_like(bl2)]).reshape(2, 1, _H)
    flags = jnp.array([[[1.0]], [[0.0]]], jnp.float32)
    n_iter = 2 - jnp.isnan(B_total[0]).astype(jnp.int32)

    def cond(st):
        return st[0] < n_iter

    def body(st):
        i, p, r, _ = st
        (segp,) = seg_fn(p, src_p, dst_p)
        p_n, r_n, h = pl.pallas_call(
            _tc_mid_body,
            out_shape=[_sds(_NPAD, _H)] * 3,
        )(segp, cntp, r,
          lax.dynamic_index_in_dim(stack_blc, i, keepdims=False),
          lax.dynamic_index_in_dim(stack_wl, i, keepdims=False),
          lax.dynamic_index_in_dim(stack_wr, i, keepdims=False),
          lax.dynamic_index_in_dim(stack_bln, i, keepdims=False),
          lax.dynamic_index_in_dim(flags, i, keepdims=False))
        return (i + 1, p_n, r_n, h)

    st0 = (jnp.int32(0), p1, r1, jnp.zeros((_NPAD, _H), jnp.float32))
    _, _, _, h = lax.while_loop(cond, body, st0)

    out = pl.pallas_call(
        _tc_c_body,
        out_shape=_sds(_NPAD, 1),
    )(h, Wm1, bm1.reshape(1, -1), Wm2, bm2.reshape(1, -1),
      batch_p.reshape(-1, 1), B_total.reshape(1, -1))
    return out[:n, 0]


def kernel(x, edge_index, edge_attr, batch, B_total,
           Wl1, bl1, Wr1, Wl2, bl2, Wr2, Wm1, bm1, Wm2, bm2):
    del edge_attr
    return _impl(x, edge_index, batch, B_total,
                 Wl1, bl1, Wr1, Wl2, bl2, Wr2, Wm1, bm1, Wm2, bm2)


# speedup vs baseline: 1.0596x; 1.0596x over previous
import functools

import jax
import jax.numpy as jnp
from jax import lax
from jax.experimental import pallas as pl
from jax.experimental.pallas import tpu as pltpu
from jax.experimental.pallas import tpu_sc as plsc

_N = 10000
_H = 64
_G = 16
_SUB = 128
_NC = 2
_NS = 16
_NW = _NC * _NS
_NPAD = 10240
_NSH = _NPAD // _NS


def _seg_inner(rpw0, rpw1, p_hbm, src_hbm, dst_hbm,
               seg_out, src_idx, dst_idx, gbuf0, gbuf1, gbuf2, gbuf3, acc,
               semg0, semg1, semg2, semg3, sems0, sems1, sems2, sems3):
    c = lax.axis_index("c")
    s = lax.axis_index("s")
    rpw = jnp.where(c == 0, rpw0, rpw1)
    base = pl.multiple_of(
        jnp.where(c == 0, s * rpw0, _NS * rpw0 + s * rpw1), 8)
    rmax = max(rpw0, rpw1)
    pltpu.sync_copy(src_hbm.at[pl.ds(base, rmax)], src_idx)
    pltpu.sync_copy(dst_hbm.at[pl.ds(base, rmax)], dst_idx)
    zv = jnp.zeros((16,), jnp.float32)

    def zrow(i, carry):
        for j in range(_H // 16):
            gbuf0[i, pl.ds(j * 16, 16)] = zv
        return carry

    lax.fori_loop(0, _SUB, zrow, 0)
    for i in range(_NSH // _SUB):
        pltpu.sync_copy(gbuf0, acc.at[pl.ds(s * _NSH + i * _SUB, _SUB)])
    plsc.subcore_barrier()

    gbufs = (gbuf0, gbuf1, gbuf2, gbuf3)
    semg = (semg0, semg1, semg2, semg3)
    sems = (sems0, sems1, sems2, sems3)
    dummy = p_hbm.at[pl.ds(0, _SUB)]

    def g_start(k, b):
        pltpu.async_copy(p_hbm.at[src_idx.at[k]], gbufs[b], semg[b])

    def g_wait(b):
        pltpu.make_async_copy(dummy, gbufs[b], semg[b]).wait()

    def s_start(k, b):
        pltpu.async_copy(gbufs[b], acc.at[dst_idx.at[k]], sems[b], add=True)

    def s_wait(b):
        pltpu.make_async_copy(dummy, gbufs[b], sems[b]).wait()

    def ops(k, b, do_swait, do_gstart):
        b2 = (b + 2) % 4
        g_wait(b)
        s_start(k, b)
        if do_swait:
            s_wait(b2)
        if do_gstart:
            g_start(k + 2, b2)

    g_start(0, 0)
    g_start(1, 1)
    ops(0, 0, False, False)
    g_start(2, 2)
    ops(1, 1, False, False)
    g_start(3, 3)
    ops(2, 2, True, True)
    ops(3, 3, True, True)

    def round_(j, carry):
        k = 4 + 4 * j
        ops(k, 0, True, True)
        ops(k + 1, 1, True, True)
        ops(k + 2, 2, True, True)
        ops(k + 3, 3, True, True)
        return carry

    lax.fori_loop(0, (rpw - 8) // 4, round_, 0)

    ops(rpw - 4, 0, True, True)
    ops(rpw - 3, 1, True, True)
    ops(rpw - 2, 2, True, False)
    ops(rpw - 1, 3, True, False)
    s_wait(2)
    s_wait(3)

    plsc.subcore_barrier()
    sl = pl.ds(s * _NSH, _NSH)
    pltpu.sync_copy(acc.at[sl], seg_out.at[c, sl])


@functools.lru_cache(maxsize=None)
def _make_seg(rpw0, rpw1):
    mesh = plsc.VectorSubcoreMesh(core_axis_name="c", subcore_axis_name="s")
    out_type = [jax.ShapeDtypeStruct((_NC, _NPAD, _H), jnp.float32)]
    rmax = max(rpw0, rpw1)
    scratch = (
        [pltpu.VMEM((rmax, _SUB), jnp.int32)] * 2
        + [pltpu.VMEM((_SUB, _H), jnp.float32)] * 4
        + [pltpu.VMEM_SHARED((_NPAD, _H), jnp.float32)]
        + [pltpu.SemaphoreType.DMA] * 8
    )

    @functools.partial(pl.kernel, out_type=out_type, mesh=mesh,
                       compiler_params=pltpu.CompilerParams(
                           use_tc_tiling_on_sc=False),
                       scratch_types=scratch)
    def seg_k(p_hbm, src_hbm, dst_hbm, seg_out,
              src_idx, dst_idx, gbuf0, gbuf1, gbuf2, gbuf3, acc,
              semg0, semg1, semg2, semg3, sems0, sems1, sems2, sems3):
        _seg_inner(rpw0, rpw1, p_hbm, src_hbm, dst_hbm,
                   seg_out, src_idx, dst_idx,
                   gbuf0, gbuf1, gbuf2, gbuf3, acc,
                   semg0, semg1, semg2, semg3,
                   sems0, sems1, sems2, sems3)

    return seg_k


@functools.lru_cache(maxsize=None)
def _make_cnt(rpw):
    mesh = plsc.VectorSubcoreMesh(core_axis_name="c", subcore_axis_name="s")

    @functools.partial(
        pl.kernel,
        out_type=[jax.ShapeDtypeStruct((_NC, _NPAD, 8), jnp.float32)],
        mesh=mesh,
        compiler_params=pltpu.CompilerParams(use_tc_tiling_on_sc=False),
        scratch_types=[
            pltpu.VMEM((rpw, _SUB), jnp.int32),
            pltpu.VMEM((_SUB, 8), jnp.float32),
            pltpu.VMEM_SHARED((_NPAD, 8), jnp.float32),
            pltpu.SemaphoreType.DMA,
        ])
    def cnt_k(dst_hbm, zc_hbm, ones_hbm, cnt_out,
              dst_idx, ones_v, cnt_acc, semc):
        c = lax.axis_index("c")
        s = lax.axis_index("s")
        base = (c * _NS + s) * rpw
        pltpu.sync_copy(dst_hbm.at[pl.ds(base, rpw)], dst_idx)
        psl = pl.ds(s * _NSH, _NSH)
        pltpu.sync_copy(zc_hbm, cnt_acc.at[psl])
        pltpu.sync_copy(ones_hbm, ones_v)
        plsc.subcore_barrier()

        def step(k, carry):
            pltpu.async_copy(ones_v, cnt_acc.at[dst_idx.at[k]], semc,
                             add=True)
            return carry

        lax.fori_loop(0, rpw, step, 0)

        def drain(i, carry):
            pltpu.make_async_copy(ones_hbm, ones_v, semc).wait()
            return carry

        lax.fori_loop(0, rpw, drain, 0)
        plsc.subcore_barrier()
        pltpu.sync_copy(cnt_acc.at[psl], cnt_out.at[c, psl])

    return cnt_k


_DNUM = (((1,), (1,)), ((), ()))


def _tc_a_body(x_ref, wl_ref, wr_ref, p_out, r_out):
    xv = x_ref[...]
    p_out[...] = lax.dot_general(xv, wl_ref[...], _DNUM,
                                 preferred_element_type=jnp.float32)
    r_out[...] = lax.dot_general(xv, wr_ref[...], _DNUM,
                                 preferred_element_type=jnp.float32)


def _tc_mid_body(segp_ref, cntp_ref, r_ref, blc_ref, wl_ref, wr_ref,
                 bln_ref, flag_ref, p_out, r_out, h_out):
    seg = segp_ref[0] + segp_ref[1]
    cnt = cntp_ref[0, :, 0:1] + cntp_ref[1, :, 0:1]
    out = seg / jnp.maximum(cnt, 1.0) + blc_ref[...] + r_ref[...]
    f = flag_ref[...]
    h = out + f * (jnp.maximum(out, 0.0) - out)
    h_out[...] = h
    p_out[...] = lax.dot_general(h, wl_ref[...], _DNUM,
                                 preferred_element_type=jnp.float32)
    r_out[...] = lax.dot_general(h, wr_ref[...], _DNUM,
                                 preferred_element_type=jnp.float32) + bln_ref[...]


def _tc_c_body(h_ref, wm1_ref, bm1_ref, wm2_ref, bm2_ref,
               batch_ref, bt_ref, out_ref):
    h = h_ref[...]
    m = jnp.maximum(lax.dot_general(h, wm1_ref[...], _DNUM,
                                    preferred_element_type=jnp.float32)
                    + bm1_ref[...], 0.0)
    z = jnp.sum(m * wm2_ref[...], axis=1, keepdims=True) + bm2_ref[...]
    pi = jax.nn.sigmoid(z)
    b = batch_ref[...]
    gid = lax.broadcasted_iota(jnp.int32, (1, _G), 1)
    onehot = (b == gid).astype(jnp.float32)
    total = jnp.sum(onehot * pi, axis=0, keepdims=True)
    ratio = jnp.minimum(bt_ref[...] / (total + 1e-12), 1.0)
    rnode = jnp.sum(onehot * ratio, axis=1, keepdims=True)
    out_ref[...] = pi * rnode


def _sds(*shape):
    return jax.ShapeDtypeStruct(shape, jnp.float32)


@jax.jit
def _impl(x, edge_index, batch, B_total,
          Wl1, bl1, Wr1, Wl2, bl2, Wr2, Wm1, bm1, Wm2, bm2):
    n, f_in = x.shape
    e = edge_index.shape[1]
    rpw = -(-e // (_NW * _SUB))
    rpw = -(-rpw // 8) * 8
    rt = rpw * _NW
    epad = rt * _SUB
    src_p = jnp.concatenate(
        [edge_index[0], jnp.zeros((epad - e,), jnp.int32)]).reshape(rt, _SUB)
    dst_p = jnp.concatenate(
        [edge_index[1], jnp.full((epad - e,), _N, jnp.int32)]).reshape(rt, _SUB)
    zc = jnp.zeros((_NSH, 8), jnp.float32)
    ones8 = jnp.ones((_SUB, 8), jnp.float32)
    x_p = jnp.pad(x, ((0, _NPAD - n), (0, 0)))
    batch_p = jnp.concatenate(
        [batch, jnp.full((_NPAD - n,), _G, jnp.int32)])

    p1, r1 = pl.pallas_call(
        _tc_a_body,
        out_shape=[_sds(_NPAD, _H), _sds(_NPAD, _H)],
    )(x_p, Wl1, Wr1)

    (cntp,) = _make_cnt(rpw)(dst_p, zc, ones8)
    rpw1 = max(8, (2 * rpw * 3 // 10) // 8 * 8)
    rpw0 = 2 * rpw - rpw1
    seg_fn = _make_seg(rpw0, rpw1)

    stack_wl = jnp.stack([Wl2, jnp.zeros_like(Wl2)])
    stack_wr = jnp.stack([Wr2, jnp.zeros_like(Wr2)])
    stack_blc = jnp.stack([bl1, bl2]).reshape(2, 1, _H)
    stack_bln = jnp.stack([bl2, jnp.zeros_like(bl2)]).reshape(2, 1, _H)
    flags = jnp.array([[[1.0]], [[0.0]]], jnp.float32)
    n_iter = 2 - jnp.isnan(B_total[0]).astype(jnp.int32)

    def cond(st):
        return st[0] < n_iter

    def body(st):
        i, p, r, _ = st
        (segp,) = seg_fn(p, src_p, dst_p)
        p_n, r_n, h = pl.pallas_call(
            _tc_mid_body,
            out_shape=[_sds(_NPAD, _H)] * 3,
        )(segp, cntp, r,
          lax.dynamic_index_in_dim(stack_blc, i, keepdims=False),
          lax.dynamic_index_in_dim(stack_wl, i, keepdims=False),
          lax.dynamic_index_in_dim(stack_wr, i, keepdims=False),
          lax.dynamic_index_in_dim(stack_bln, i, keepdims=False),
          lax.dynamic_index_in_dim(flags, i, keepdims=False))
        return (i + 1, p_n, r_n, h)

    st0 = (jnp.int32(0), p1, r1, jnp.zeros((_NPAD, _H), jnp.float32))
    _, _, _, h = lax.while_loop(cond, body, st0)

    out = pl.pallas_call(
        _tc_c_body,
        out_shape=_sds(_NPAD, 1),
    )(h, Wm1, bm1.reshape(1, -1), Wm2, bm2.reshape(1, -1),
      batch_p.reshape(-1, 1), B_total.reshape(1, -1))
    return out[:n, 0]


def kernel(x, edge_index, edge_attr, batch, B_total,
           Wl1, bl1, Wr1, Wl2, bl2, Wr2, Wm1, bm1, Wm2, bm2):
    del edge_attr
    return _impl(x, edge_index, batch, B_total,
                 Wl1, bl1, Wr1, Wl2, bl2, Wr2, Wm1, bm1, Wm2, bm2)
